# Initial kernel scaffold; baseline (speedup 1.0000x reference)
import jax
import jax.numpy as jnp
from jax.experimental import pallas as pl


def kernel(x, Wr_q, Wr_k, Wr_v, We_q, be_q, We_k, be_k, We_v, be_v, Wo):
    raise NotImplementedError("write your pallas kernel here")


# speedup vs baseline: 1.4794x; 1.4794x over previous
import functools

import jax
import jax.numpy as jnp
import numpy as np
from jax.experimental import pallas as pl
from jax.experimental.pallas import tpu as pltpu

_B, _T, _H, _D, _E, _K = 1, 2048, 16, 64, 8, 2
_C = _H * _D
_SCALE = 1.0 / np.sqrt(_D)
_BQ = 256
_BM = 256


def _route_project(xb, wr_ref, wef_ref, be_ref, out_ref, l_ref):
    logits = jnp.dot(xb, wr_ref[...], preferred_element_type=jnp.float32)
    l_ref[0] = logits
    m = jnp.max(logits, axis=-1, keepdims=True)
    ex = jnp.exp(logits - m)
    p = ex / jnp.sum(ex, axis=-1, keepdims=True)
    p1 = jnp.max(p, axis=-1, keepdims=True)
    i1 = jnp.argmax(p, axis=-1)
    eidx = jax.lax.broadcasted_iota(jnp.int32, (_T, _E), 1)
    not_first = eidx != i1[:, None]
    p2 = jnp.max(jnp.where(not_first, p, -1.0), axis=-1, keepdims=True)
    sel = p >= p2
    wd = jnp.where(sel, p, 0.0) / (p1 + p2)
    y = jnp.dot(xb, wef_ref[...], preferred_element_type=jnp.float32)
    acc = jnp.dot(wd, be_ref[...], preferred_element_type=jnp.float32)
    for e in range(_E):
        acc = acc + wd[:, e : e + 1] * y[:, e * _D : (e + 1) * _D]
    out_ref[0] = acc


def _proj_kernel(xh_ref,
                 wrq_ref, wrk_ref, wrv_ref,
                 weq_ref, wek_ref, wev_ref,
                 beq_ref, bek_ref, bev_ref,
                 q_ref, k_ref, v_ref, ql_ref, kl_ref, vl_ref):
    xb = xh_ref[0]
    _route_project(xb, wrq_ref, weq_ref, beq_ref, q_ref, ql_ref)
    _route_project(xb, wrk_ref, wek_ref, bek_ref, k_ref, kl_ref)
    _route_project(xb, wrv_ref, wev_ref, bev_ref, v_ref, vl_ref)


def _attn_kernel(q_ref, k_ref, v_ref, o_ref):
    qi = pl.program_id(1)
    q = q_ref[0] * _SCALE
    m0 = jnp.full((_BQ, 1), -jnp.inf, jnp.float32)
    l0 = jnp.zeros((_BQ, 1), jnp.float32)
    a0 = jnp.zeros((_BQ, _D), jnp.float32)

    def body(j, carry):
        m, l, acc = carry
        kb = k_ref[0, pl.ds(j * _BQ, _BQ), :]
        vb = v_ref[0, pl.ds(j * _BQ, _BQ), :]
        s = jax.lax.dot_general(q, kb, (((1,), (1,)), ((), ())),
                                preferred_element_type=jnp.float32)
        ti = jax.lax.broadcasted_iota(jnp.int32, (_BQ, _BQ), 0) + qi * _BQ
        si = jax.lax.broadcasted_iota(jnp.int32, (_BQ, _BQ), 1) + j * _BQ
        s = jnp.where(si <= ti, s, -jnp.inf)
        mj = jnp.maximum(m, jnp.max(s, axis=-1, keepdims=True))
        palpha = jnp.exp(m - mj)
        pj = jnp.exp(s - mj)
        l = l * palpha + jnp.sum(pj, axis=-1, keepdims=True)
        acc = acc * palpha + jnp.dot(pj, vb, preferred_element_type=jnp.float32)
        return mj, l, acc

    m, l, acc = jax.lax.fori_loop(0, qi + 1, body, (m0, l0, a0))
    o_ref[0] = acc / l


def _oproj_kernel(o_ref, wot_ref, out_ref):
    out_ref[...] = jnp.dot(o_ref[...], wot_ref[...],
                           preferred_element_type=jnp.float32)


@functools.partial(jax.jit, static_argnames=("interpret",))
def kernel(x, Wr_q, Wr_k, Wr_v, We_q, be_q, We_k, be_k, We_v, be_v, Wo,
           interpret=False):
    xh = x.reshape(_T, _H, _D).transpose(1, 0, 2)
    wef_q = We_q.transpose(1, 0, 2).reshape(_D, _E * _D)
    wef_k = We_k.transpose(1, 0, 2).reshape(_D, _E * _D)
    wef_v = We_v.transpose(1, 0, 2).reshape(_D, _E * _D)

    full = lambda shape: pl.BlockSpec(shape, lambda h: (0,) * len(shape))
    head_blk = pl.BlockSpec((1, _T, _D), lambda h: (h, 0, 0))
    logit_blk = pl.BlockSpec((1, _T, _E), lambda h: (h, 0, 0))

    q, k, v, ql, kl, vl = pl.pallas_call(
        _proj_kernel,
        grid=(_H,),
        in_specs=[
            head_blk,
            full((_D, _E)), full((_D, _E)), full((_D, _E)),
            full((_D, _E * _D)), full((_D, _E * _D)), full((_D, _E * _D)),
            full((_E, _D)), full((_E, _D)), full((_E, _D)),
        ],
        out_specs=[head_blk, head_blk, head_blk,
                   logit_blk, logit_blk, logit_blk],
        out_shape=[
            jax.ShapeDtypeStruct((_H, _T, _D), jnp.float32),
            jax.ShapeDtypeStruct((_H, _T, _D), jnp.float32),
            jax.ShapeDtypeStruct((_H, _T, _D), jnp.float32),
            jax.ShapeDtypeStruct((_H, _T, _E), jnp.float32),
            jax.ShapeDtypeStruct((_H, _T, _E), jnp.float32),
            jax.ShapeDtypeStruct((_H, _T, _E), jnp.float32),
        ],
        interpret=interpret,
    )(xh, Wr_q, Wr_k, Wr_v, wef_q, wef_k, wef_v, be_q, be_k, be_v)

    o = pl.pallas_call(
        _attn_kernel,
        grid=(_H, _T // _BQ),
        in_specs=[
            pl.BlockSpec((1, _BQ, _D), lambda h, i: (h, i, 0)),
            pl.BlockSpec((1, _T, _D), lambda h, i: (h, 0, 0)),
            pl.BlockSpec((1, _T, _D), lambda h, i: (h, 0, 0)),
        ],
        out_specs=pl.BlockSpec((1, _BQ, _D), lambda h, i: (h, i, 0)),
        out_shape=jax.ShapeDtypeStruct((_H, _T, _D), jnp.float32),
        interpret=interpret,
    )(q, k, v)

    o_flat = o.transpose(1, 0, 2).reshape(_T, _C)
    out = pl.pallas_call(
        _oproj_kernel,
        grid=(_T // _BM,),
        in_specs=[
            pl.BlockSpec((_BM, _C), lambda i: (i, 0)),
            pl.BlockSpec((_C, _C), lambda i: (0, 0)),
        ],
        out_specs=pl.BlockSpec((_BM, _C), lambda i: (i, 0)),
        out_shape=jax.ShapeDtypeStruct((_T, _C), jnp.float32),
        interpret=interpret,
    )(o_flat, Wo.T)

    tr = lambda a: a.transpose(1, 0, 2)[None]
    return out[None], tr(ql), tr(kl), tr(vl)


# speedup vs baseline: 1.5017x; 1.0151x over previous
import functools

import jax
import jax.numpy as jnp
import numpy as np
from jax.experimental import pallas as pl
from jax.experimental.pallas import tpu as pltpu

_B, _T, _H, _D, _E, _K = 1, 2048, 16, 64, 8, 2
_C = _H * _D
_SCALE = 1.0 / np.sqrt(_D)
_BQ = 256
_BM = 256


def _route_project(xb, xb_bf, wr_ref, wef_ref, be_ref, out_ref, l_ref):
    logits = jnp.dot(xb, wr_ref[...], preferred_element_type=jnp.float32)
    l_ref[0] = logits
    m = jnp.max(logits, axis=-1, keepdims=True)
    ex = jnp.exp(logits - m)
    p = ex / jnp.sum(ex, axis=-1, keepdims=True)
    p1 = jnp.max(p, axis=-1, keepdims=True)
    i1 = jnp.argmax(p, axis=-1)
    eidx = jax.lax.broadcasted_iota(jnp.int32, (_T, _E), 1)
    not_first = eidx != i1[:, None]
    p2 = jnp.max(jnp.where(not_first, p, -1.0), axis=-1, keepdims=True)
    sel = p >= p2
    wd = jnp.where(sel, p, 0.0) / (p1 + p2)
    y = jnp.dot(xb_bf, wef_ref[...], preferred_element_type=jnp.float32)
    acc = jnp.dot(wd, be_ref[...], preferred_element_type=jnp.float32)
    for e in range(_E):
        acc = acc + wd[:, e : e + 1] * y[:, e * _D : (e + 1) * _D]
    out_ref[0] = acc.astype(jnp.bfloat16)


def _proj_kernel(xh_ref,
                 wrq_ref, wrk_ref, wrv_ref,
                 weq_ref, wek_ref, wev_ref,
                 beq_ref, bek_ref, bev_ref,
                 q_ref, k_ref, v_ref, ql_ref, kl_ref, vl_ref):
    xb = xh_ref[0]
    xb_bf = xb.astype(jnp.bfloat16)
    _route_project(xb, xb_bf, wrq_ref, weq_ref, beq_ref, q_ref, ql_ref)
    _route_project(xb, xb_bf, wrk_ref, wek_ref, bek_ref, k_ref, kl_ref)
    _route_project(xb, xb_bf, wrv_ref, wev_ref, bev_ref, v_ref, vl_ref)


def _attn_kernel(q_ref, k_ref, v_ref, o_ref):
    qi = pl.program_id(1)
    q = q_ref[0]
    m0 = jnp.full((_BQ, 1), -jnp.inf, jnp.float32)
    l0 = jnp.zeros((_BQ, 1), jnp.float32)
    a0 = jnp.zeros((_BQ, _D), jnp.float32)

    def body(j, carry):
        m, l, acc = carry
        kb = k_ref[0, pl.ds(j * _BQ, _BQ), :]
        vb = v_ref[0, pl.ds(j * _BQ, _BQ), :]
        s = jax.lax.dot_general(q, kb, (((1,), (1,)), ((), ())),
                                preferred_element_type=jnp.float32) * _SCALE
        ti = jax.lax.broadcasted_iota(jnp.int32, (_BQ, _BQ), 0) + qi * _BQ
        si = jax.lax.broadcasted_iota(jnp.int32, (_BQ, _BQ), 1) + j * _BQ
        s = jnp.where(si <= ti, s, -jnp.inf)
        mj = jnp.maximum(m, jnp.max(s, axis=-1, keepdims=True))
        palpha = jnp.exp(m - mj)
        pj = jnp.exp(s - mj).astype(jnp.bfloat16)
        l = l * palpha + jnp.sum(pj.astype(jnp.float32), axis=-1, keepdims=True)
        acc = acc * palpha + jnp.dot(pj, vb, preferred_element_type=jnp.float32)
        return mj, l, acc

    m, l, acc = jax.lax.fori_loop(0, qi + 1, body, (m0, l0, a0))
    o_ref[0] = (acc / l).astype(jnp.bfloat16)


def _oproj_kernel(o_ref, wo_ref, out_ref):
    out_ref[...] = jax.lax.dot_general(
        o_ref[...], wo_ref[...], (((1,), (1,)), ((), ())),
        preferred_element_type=jnp.float32)


@functools.partial(jax.jit, static_argnames=("interpret",))
def kernel(x, Wr_q, Wr_k, Wr_v, We_q, be_q, We_k, be_k, We_v, be_v, Wo,
           interpret=False):
    xh = x.reshape(_T, _H, _D).transpose(1, 0, 2)
    bf = jnp.bfloat16
    wef_q = We_q.transpose(1, 0, 2).reshape(_D, _E * _D).astype(bf)
    wef_k = We_k.transpose(1, 0, 2).reshape(_D, _E * _D).astype(bf)
    wef_v = We_v.transpose(1, 0, 2).reshape(_D, _E * _D).astype(bf)

    full = lambda shape: pl.BlockSpec(shape, lambda h: (0,) * len(shape))
    head_blk = pl.BlockSpec((1, _T, _D), lambda h: (h, 0, 0))
    logit_blk = pl.BlockSpec((1, _T, _E), lambda h: (h, 0, 0))

    q, k, v, ql, kl, vl = pl.pallas_call(
        _proj_kernel,
        grid=(_H,),
        in_specs=[
            head_blk,
            full((_D, _E)), full((_D, _E)), full((_D, _E)),
            full((_D, _E * _D)), full((_D, _E * _D)), full((_D, _E * _D)),
            full((_E, _D)), full((_E, _D)), full((_E, _D)),
        ],
        out_specs=[head_blk, head_blk, head_blk,
                   logit_blk, logit_blk, logit_blk],
        out_shape=[
            jax.ShapeDtypeStruct((_H, _T, _D), bf),
            jax.ShapeDtypeStruct((_H, _T, _D), bf),
            jax.ShapeDtypeStruct((_H, _T, _D), bf),
            jax.ShapeDtypeStruct((_H, _T, _E), jnp.float32),
            jax.ShapeDtypeStruct((_H, _T, _E), jnp.float32),
            jax.ShapeDtypeStruct((_H, _T, _E), jnp.float32),
        ],
        interpret=interpret,
    )(xh, Wr_q, Wr_k, Wr_v, wef_q, wef_k, wef_v, be_q, be_k, be_v)

    o = pl.pallas_call(
        _attn_kernel,
        grid=(_H, _T // _BQ),
        in_specs=[
            pl.BlockSpec((1, _BQ, _D), lambda h, i: (h, i, 0)),
            pl.BlockSpec((1, _T, _D), lambda h, i: (h, 0, 0)),
            pl.BlockSpec((1, _T, _D), lambda h, i: (h, 0, 0)),
        ],
        out_specs=pl.BlockSpec((1, _BQ, _D), lambda h, i: (h, i, 0)),
        out_shape=jax.ShapeDtypeStruct((_H, _T, _D), bf),
        interpret=interpret,
    )(q, k, v)

    o_flat = o.transpose(1, 0, 2).reshape(_T, _C)
    out = pl.pallas_call(
        _oproj_kernel,
        grid=(_T // _BM,),
        in_specs=[
            pl.BlockSpec((_BM, _C), lambda i: (i, 0)),
            pl.BlockSpec((_C, _C), lambda i: (0, 0)),
        ],
        out_specs=pl.BlockSpec((_BM, _C), lambda i: (i, 0)),
        out_shape=jax.ShapeDtypeStruct((_T, _C), jnp.float32),
        interpret=interpret,
    )(o_flat, Wo.astype(bf))

    tr = lambda a: a.transpose(1, 0, 2)[None]
    return out[None], tr(ql), tr(kl), tr(vl)


# speedup vs baseline: 2.5592x; 1.7042x over previous
import functools

import jax
import jax.numpy as jnp
import numpy as np
from jax.experimental import pallas as pl
from jax.experimental.pallas import tpu as pltpu

_B, _T, _H, _D, _E, _K = 1, 2048, 16, 64, 8, 2
_C = _H * _D
_SCALE = 1.0 / np.sqrt(_D)
_BQ = 512
_BM = 256


def _route_project(xb, xb_bf, wr_ref, wef_ref, be_ref, out_ref, l_ref):
    logits = jnp.dot(xb, wr_ref[...], preferred_element_type=jnp.float32)
    l_ref[0] = logits
    m = jnp.max(logits, axis=-1, keepdims=True)
    ex = jnp.exp(logits - m)
    p = ex / jnp.sum(ex, axis=-1, keepdims=True)
    p1 = jnp.max(p, axis=-1, keepdims=True)
    i1 = jnp.argmax(p, axis=-1)
    eidx = jax.lax.broadcasted_iota(jnp.int32, (_T, _E), 1)
    not_first = eidx != i1[:, None]
    p2 = jnp.max(jnp.where(not_first, p, -1.0), axis=-1, keepdims=True)
    sel = p >= p2
    wd = jnp.where(sel, p, 0.0) / (p1 + p2)
    y = jnp.dot(xb_bf, wef_ref[...], preferred_element_type=jnp.float32)
    ci = jax.lax.broadcasted_iota(jnp.int32, (_E, _E * _D), 1)
    ei = jax.lax.broadcasted_iota(jnp.int32, (_E, _E * _D), 0)
    s_mat = (ci // _D == ei).astype(jnp.float32)
    wdexp = jnp.dot(wd, s_mat, preferred_element_type=jnp.float32)
    z = (wdexp * (y + be_ref[...])).astype(jnp.bfloat16)
    fi = jax.lax.broadcasted_iota(jnp.int32, (_E * _D, _D), 0)
    fj = jax.lax.broadcasted_iota(jnp.int32, (_E * _D, _D), 1)
    f_mat = (fi % _D == fj).astype(jnp.bfloat16)
    acc = jnp.dot(z, f_mat, preferred_element_type=jnp.float32)
    out_ref[0] = acc.astype(jnp.bfloat16)


def _proj_kernel(xh_ref,
                 wrq_ref, wrk_ref, wrv_ref,
                 weq_ref, wek_ref, wev_ref,
                 beq_ref, bek_ref, bev_ref,
                 q_ref, k_ref, v_ref, ql_ref, kl_ref, vl_ref):
    xb = xh_ref[0]
    xb_bf = xb.astype(jnp.bfloat16)
    _route_project(xb, xb_bf, wrq_ref, weq_ref, beq_ref, q_ref, ql_ref)
    _route_project(xb, xb_bf, wrk_ref, wek_ref, bek_ref, k_ref, kl_ref)
    _route_project(xb, xb_bf, wrv_ref, wev_ref, bev_ref, v_ref, vl_ref)


def _attn_kernel(q_ref, k_ref, v_ref, o_ref):
    qi = pl.program_id(1)
    q = q_ref[0]
    m0 = jnp.full((_BQ, 1), -jnp.inf, jnp.float32)
    l0 = jnp.zeros((_BQ, 1), jnp.float32)
    a0 = jnp.zeros((_BQ, _D), jnp.float32)

    def body(j, carry):
        m, l, acc = carry
        kb = k_ref[0, pl.ds(j * _BQ, _BQ), :]
        vb = v_ref[0, pl.ds(j * _BQ, _BQ), :]
        s = jax.lax.dot_general(q, kb, (((1,), (1,)), ((), ())),
                                preferred_element_type=jnp.float32) * _SCALE
        ti = jax.lax.broadcasted_iota(jnp.int32, (_BQ, _BQ), 0) + qi * _BQ
        si = jax.lax.broadcasted_iota(jnp.int32, (_BQ, _BQ), 1) + j * _BQ
        s = jnp.where(si <= ti, s, -jnp.inf)
        mj = jnp.maximum(m, jnp.max(s, axis=-1, keepdims=True))
        palpha = jnp.exp(m - mj)
        pj = jnp.exp(s - mj).astype(jnp.bfloat16)
        l = l * palpha + jnp.sum(pj.astype(jnp.float32), axis=-1, keepdims=True)
        acc = acc * palpha + jnp.dot(pj, vb, preferred_element_type=jnp.float32)
        return mj, l, acc

    m, l, acc = jax.lax.fori_loop(0, qi + 1, body, (m0, l0, a0))
    o_ref[0] = (acc / l).astype(jnp.bfloat16)


def _oproj_kernel(o_ref, wo_ref, out_ref):
    out_ref[...] = jax.lax.dot_general(
        o_ref[...], wo_ref[...], (((1,), (1,)), ((), ())),
        preferred_element_type=jnp.float32)


@functools.partial(jax.jit, static_argnames=("interpret",))
def kernel(x, Wr_q, Wr_k, Wr_v, We_q, be_q, We_k, be_k, We_v, be_v, Wo,
           interpret=False):
    xh = x.reshape(_T, _H, _D).transpose(1, 0, 2)
    bf = jnp.bfloat16
    wef_q = We_q.transpose(1, 0, 2).reshape(_D, _E * _D).astype(bf)
    wef_k = We_k.transpose(1, 0, 2).reshape(_D, _E * _D).astype(bf)
    wef_v = We_v.transpose(1, 0, 2).reshape(_D, _E * _D).astype(bf)

    full = lambda shape: pl.BlockSpec(shape, lambda h: (0,) * len(shape))
    head_blk = pl.BlockSpec((1, _T, _D), lambda h: (h, 0, 0))
    logit_blk = pl.BlockSpec((1, _T, _E), lambda h: (h, 0, 0))

    q, k, v, ql, kl, vl = pl.pallas_call(
        _proj_kernel,
        grid=(_H,),
        in_specs=[
            head_blk,
            full((_D, _E)), full((_D, _E)), full((_D, _E)),
            full((_D, _E * _D)), full((_D, _E * _D)), full((_D, _E * _D)),
            full((1, _E * _D)), full((1, _E * _D)), full((1, _E * _D)),
        ],
        out_specs=[head_blk, head_blk, head_blk,
                   logit_blk, logit_blk, logit_blk],
        out_shape=[
            jax.ShapeDtypeStruct((_H, _T, _D), bf),
            jax.ShapeDtypeStruct((_H, _T, _D), bf),
            jax.ShapeDtypeStruct((_H, _T, _D), bf),
            jax.ShapeDtypeStruct((_H, _T, _E), jnp.float32),
            jax.ShapeDtypeStruct((_H, _T, _E), jnp.float32),
            jax.ShapeDtypeStruct((_H, _T, _E), jnp.float32),
        ],
        interpret=interpret,
    )(xh, Wr_q, Wr_k, Wr_v, wef_q, wef_k, wef_v,
      be_q.reshape(1, _E * _D), be_k.reshape(1, _E * _D),
      be_v.reshape(1, _E * _D))

    o = pl.pallas_call(
        _attn_kernel,
        grid=(_H, _T // _BQ),
        in_specs=[
            pl.BlockSpec((1, _BQ, _D), lambda h, i: (h, i, 0)),
            pl.BlockSpec((1, _T, _D), lambda h, i: (h, 0, 0)),
            pl.BlockSpec((1, _T, _D), lambda h, i: (h, 0, 0)),
        ],
        out_specs=pl.BlockSpec((1, _BQ, _D), lambda h, i: (h, i, 0)),
        out_shape=jax.ShapeDtypeStruct((_H, _T, _D), bf),
        interpret=interpret,
    )(q, k, v)

    o_flat = o.transpose(1, 0, 2).reshape(_T, _C)
    out = pl.pallas_call(
        _oproj_kernel,
        grid=(_T // _BM,),
        in_specs=[
            pl.BlockSpec((_BM, _C), lambda i: (i, 0)),
            pl.BlockSpec((_C, _C), lambda i: (0, 0)),
        ],
        out_specs=pl.BlockSpec((_BM, _C), lambda i: (i, 0)),
        out_shape=jax.ShapeDtypeStruct((_T, _C), jnp.float32),
        interpret=interpret,
    )(o_flat, Wo.astype(bf))

    tr = lambda a: a.transpose(1, 0, 2)[None]
    return out[None], tr(ql), tr(kl), tr(vl)
